# baseline (speedup 1.0000x reference)
import functools

import jax
import jax.numpy as jnp
from jax import lax
from jax.experimental import pallas as pl
from jax.experimental.pallas import tpu as pltpu
from jax.experimental.pallas import tpu_sc as plsc

NC = 2
NS = 16
TILES = NC * NS
LANES = 16

K = 16
EG = 2048
RG = 8


def _s1_body(x_ref, wall_ref, wd_ref, rec_ref, ad_ref):
    x = x_ref[...]
    rec_ref[...] = jnp.dot(x, wall_ref[...], preferred_element_type=jnp.float32)
    ad_ref[...] = jnp.dot(x, wd_ref[...], preferred_element_type=jnp.float32)


def _s3_body(acc_ref, deg_ref, wg_ref, b_ref, rec2_ref, dinv_ref):
    h = jnp.maximum(acc_ref[...] + b_ref[...], 0.0)
    dinv = lax.rsqrt(deg_ref[...])
    mm = jnp.dot(h, wg_ref[...], preferred_element_type=jnp.float32)
    rec2_ref[...] = mm * dinv
    dinv_ref[...] = dinv


def _s6_body(pooled_ref, cell_ref,
             g1_ref, g1b_ref, g2_ref, g2b_ref, g3_ref, g3b_ref,
             r1_ref, r1b_ref, r2_ref, r2b_ref, r3_ref, r3b_ref,
             f1_ref, f1b_ref, f2_ref, f2b_ref, ow_ref, ob_ref,
             out_ref):
    f32 = jnp.float32
    y = jnp.maximum(jnp.dot(pooled_ref[...], g1_ref[...], preferred_element_type=f32)
                    + g1b_ref[...], 0.0)
    y = jnp.dot(y, g2_ref[...], preferred_element_type=f32) + g2b_ref[...]
    y = jnp.dot(y, g3_ref[...], preferred_element_type=f32) + g3b_ref[...]
    nb = out_ref.shape[0]
    h1 = y[:nb]
    h2 = y[nb:2 * nb]
    cv = cell_ref[...]
    nrm = jnp.sqrt(jnp.sum(cv * cv, axis=1, keepdims=True))
    cv = cv / jnp.maximum(nrm, 1e-12)
    cv = jnp.maximum(jnp.dot(cv, r1_ref[...], preferred_element_type=f32)
                     + r1b_ref[...], 0.0)
    cv = jnp.maximum(jnp.dot(cv, r2_ref[...], preferred_element_type=f32)
                     + r2b_ref[...], 0.0)
    cv = jnp.dot(cv, r3_ref[...], preferred_element_type=f32) + r3b_ref[...]
    xc = jnp.concatenate([h1, h2, cv], axis=1)
    xc = jnp.maximum(jnp.dot(xc, f1_ref[...], preferred_element_type=f32)
                     + f1b_ref[...], 0.0)
    xc = jnp.maximum(jnp.dot(xc, f2_ref[...], preferred_element_type=f32)
                     + f2b_ref[...], 0.0)
    out_ref[...] = jnp.dot(xc, ow_ref[...], preferred_element_type=f32) + ob_ref[...]


def _make_gat_agg(N1, NPT, REC1, H, CP):
    mesh = plsc.VectorSubcoreMesh(core_axis_name="c", subcore_axis_name="s")

    @functools.partial(
        pl.kernel, mesh=mesh,
        compiler_params=pltpu.CompilerParams(use_tc_tiling_on_sc=False),
        out_type=jax.ShapeDtypeStruct((N1, REC1), jnp.float32),
        scratch_types=[
            pltpu.VMEM((EG + 16,), jnp.int32),
            pltpu.VMEM((EG + 16,), jnp.int32),
            pltpu.VMEM((K, REC1), jnp.float32),
            pltpu.VMEM((NPT, 16), jnp.float32),
            pltpu.VMEM((NPT + 16,), jnp.int32),
            pltpu.VMEM((REC1,), jnp.float32),
            pltpu.VMEM((16,), jnp.float32),
            pltpu.VMEM((RG, REC1), jnp.float32),
            pltpu.SemaphoreType.DMA,
        ])
    def gat_agg(rec_hbm, ad_hbm, rp_hbm, src_hbm, dst_hbm, out_hbm,
                srcv, dstv, gbuf, adv, rpv, acc, wsum, stage, gsem):
        t = lax.axis_index("s") * NC + lax.axis_index("c")
        ns = pl.multiple_of(t * NPT, 8)
        pltpu.sync_copy(rp_hbm.at[pl.ds(ns, NPT + 8)], rpv.at[pl.ds(0, NPT + 8)])
        pltpu.sync_copy(ad_hbm.at[pl.ds(ns, NPT)], adv)
        e0 = rpv[pl.ds(0, 16)][0]
        e1 = rpv[pl.ds(NPT, 16)][0]
        a0 = pl.multiple_of((e0 // K) * K, 8)
        ngr = (e1 - a0 + (EG - 1)) // EG

        zero16 = jnp.zeros((16,), jnp.float32)
        for q in range(REC1 // 16):
            acc[pl.ds(q * 16, 16)] = zero16
        wsum[...] = zero16

        def flush(cur):
            r = cur - ns
            slot = r - (r // RG) * RG
            ws = wsum[...]
            for hh in range(H):
                s = ws[hh] + 1e-16
                for j5 in range(CP // 16):
                    off = 16 + hh * CP + j5 * 16
                    stage[slot, pl.ds(off, 16)] = acc[pl.ds(off, 16)] / s
            stage[slot, pl.ds(0, 16)] = zero16

            @pl.when(slot == RG - 1)
            def _():
                pltpu.sync_copy(
                    stage, out_hbm.at[pl.ds(pl.multiple_of(cur - (RG - 1), 8), RG)])

        def edge_body(j, carry):
            cur, ci = carry
            d = dstv[pl.ds(ci * K + j, 16)][0]
            trans = d != cur

            @pl.when(jnp.logical_and(
                trans, jnp.logical_and(cur >= ns, cur < ns + NPT)))
            def _():
                flush(cur)

            @pl.when(trans)
            def _():
                for q in range(REC1 // 16):
                    acc[pl.ds(q * 16, 16)] = zero16
                wsum[...] = zero16

            cur = jnp.where(trans, d, cur)
            ia = jnp.minimum(d - ns, NPT - 1)
            av = gbuf[j, pl.ds(0, 16)] + adv[ia]
            al = jnp.maximum(av, 0.2 * av)
            w = jnp.exp(al)
            wsum[...] = wsum[...] + w
            for hh in range(H):
                wh = w[hh]
                for j5 in range(CP // 16):
                    off = 16 + hh * CP + j5 * 16
                    plsc.addupdate(acc.at[pl.ds(off, 16)],
                                   wh * gbuf[j, pl.ds(off, 16)])
            return cur, ci

        def group_body(g, cur):
            gstart = pl.multiple_of(a0 + g * EG, 8)
            pltpu.sync_copy(src_hbm.at[pl.ds(gstart, EG)], srcv.at[pl.ds(0, EG)])
            pltpu.sync_copy(dst_hbm.at[pl.ds(gstart, EG)], dstv.at[pl.ds(0, EG)])
            ncg = jnp.minimum((e1 - gstart + (K - 1)) // K, EG // K)

            def chunk(ci, cur):
                idxv = srcv[pl.ds(ci * K, K)]
                pltpu.async_copy(rec_hbm.at[idxv], gbuf, gsem).wait()
                cur, _ = lax.fori_loop(0, K, edge_body, (cur, ci))
                return cur

            return lax.fori_loop(0, ncg, chunk, cur)

        cur = lax.fori_loop(0, ngr, group_body, jnp.int32(-1))

        @pl.when(jnp.logical_and(cur >= ns, cur < ns + NPT))
        def _():
            flush(cur)

    return gat_agg


def _make_gcn_agg(N1, NPT, REC2):
    mesh = plsc.VectorSubcoreMesh(core_axis_name="c", subcore_axis_name="s")

    @functools.partial(
        pl.kernel, mesh=mesh,
        compiler_params=pltpu.CompilerParams(use_tc_tiling_on_sc=False),
        out_type=jax.ShapeDtypeStruct((N1, REC2), jnp.float32),
        scratch_types=[
            pltpu.VMEM((EG + 16,), jnp.int32),
            pltpu.VMEM((EG + 16,), jnp.int32),
            pltpu.VMEM((K, REC2), jnp.float32),
            pltpu.VMEM((NPT + 16,), jnp.float32),
            pltpu.VMEM((NPT + 16,), jnp.int32),
            pltpu.VMEM((REC2,), jnp.float32),
            pltpu.VMEM((RG, REC2), jnp.float32),
            pltpu.SemaphoreType.DMA,
        ])
    def gcn_agg(rec_hbm, dinv_hbm, rp_hbm, src_hbm, dst_hbm, out_hbm,
                srcv, dstv, gbuf, dvv, rpv, acc, stage, gsem):
        t = lax.axis_index("s") * NC + lax.axis_index("c")
        ns = pl.multiple_of(t * NPT, 8)
        pltpu.sync_copy(rp_hbm.at[pl.ds(ns, NPT + 8)], rpv.at[pl.ds(0, NPT + 8)])
        pltpu.sync_copy(dinv_hbm.at[pl.ds(ns, NPT)], dvv.at[pl.ds(0, NPT)])
        e0 = rpv[pl.ds(0, 16)][0]
        e1 = rpv[pl.ds(NPT, 16)][0]
        a0 = pl.multiple_of((e0 // K) * K, 8)
        ngr = (e1 - a0 + (EG - 1)) // EG

        zero16 = jnp.zeros((16,), jnp.float32)
        for q in range(REC2 // 16):
            acc[pl.ds(q * 16, 16)] = zero16

        def flush(cur):
            r = cur - ns
            slot = r - (r // RG) * RG
            for q in range(REC2 // 16):
                stage[slot, pl.ds(q * 16, 16)] = acc[pl.ds(q * 16, 16)]

            @pl.when(slot == RG - 1)
            def _():
                pltpu.sync_copy(
                    stage, out_hbm.at[pl.ds(pl.multiple_of(cur - (RG - 1), 8), RG)])

        def edge_body(j, carry):
            cur, ci = carry
            d = dstv[pl.ds(ci * K + j, 16)][0]
            trans = d != cur

            @pl.when(jnp.logical_and(
                trans, jnp.logical_and(cur >= ns, cur < ns + NPT)))
            def _():
                flush(cur)

            @pl.when(trans)
            def _():
                for q in range(REC2 // 16):
                    acc[pl.ds(q * 16, 16)] = zero16

            cur = jnp.where(trans, d, cur)
            iv = jnp.minimum(d - ns, NPT - 1)
            dv = dvv[pl.ds(iv, 16)][0]
            for q in range(REC2 // 16):
                plsc.addupdate(acc.at[pl.ds(q * 16, 16)],
                               dv * gbuf[j, pl.ds(q * 16, 16)])
            return cur, ci

        def group_body(g, cur):
            gstart = pl.multiple_of(a0 + g * EG, 8)
            pltpu.sync_copy(src_hbm.at[pl.ds(gstart, EG)], srcv.at[pl.ds(0, EG)])
            pltpu.sync_copy(dst_hbm.at[pl.ds(gstart, EG)], dstv.at[pl.ds(0, EG)])
            ncg = jnp.minimum((e1 - gstart + (K - 1)) // K, EG // K)

            def chunk(ci, cur):
                idxv = srcv[pl.ds(ci * K, K)]
                pltpu.async_copy(rec_hbm.at[idxv], gbuf, gsem).wait()
                cur, _ = lax.fori_loop(0, K, edge_body, (cur, ci))
                return cur

            return lax.fori_loop(0, ncg, chunk, cur)

        cur = lax.fori_loop(0, ngr, group_body, jnp.int32(-1))

        @pl.when(jnp.logical_and(cur >= ns, cur < ns + NPT))
        def _():
            flush(cur)

    return gcn_agg


def _make_pool(N1, BT, REC2):
    mesh = plsc.VectorSubcoreMesh(core_axis_name="c", subcore_axis_name="s")
    SEGT = BT // TILES
    NR = 8

    @functools.partial(
        pl.kernel, mesh=mesh,
        compiler_params=pltpu.CompilerParams(use_tc_tiling_on_sc=False),
        out_type=jax.ShapeDtypeStruct((BT, 2 * REC2), jnp.float32),
        scratch_types=[
            pltpu.VMEM((SEGT + 24,), jnp.int32),
            pltpu.VMEM((NR, REC2), jnp.float32),
            pltpu.VMEM((REC2,), jnp.float32),
            pltpu.VMEM((REC2,), jnp.float32),
            pltpu.VMEM((REC2,), jnp.float32),
            pltpu.VMEM((SEGT, 2 * REC2), jnp.float32),
        ])
    def pool(acc2_hbm, b_hbm, bp_hbm, out_hbm,
             bpv, grp, bv, maxa, suma, pstage):
        t = lax.axis_index("s") * NC + lax.axis_index("c")
        s0 = pl.multiple_of(t * SEGT, 8)
        pltpu.sync_copy(bp_hbm.at[pl.ds(s0, SEGT + 8)], bpv.at[pl.ds(0, SEGT + 8)])
        pltpu.sync_copy(b_hbm, bv)
        zero16 = jnp.zeros((16,), jnp.float32)

        def seg_body(s, _):
            n0 = bpv[pl.ds(s, 16)][0]
            n1 = bpv[pl.ds(s + 1, 16)][0]
            for q in range(REC2 // 16):
                maxa[pl.ds(q * 16, 16)] = zero16
                suma[pl.ds(q * 16, 16)] = zero16

            ka0 = n0 // NR

            def chunk_body(kk, _):
                base = pl.multiple_of((ka0 + kk) * NR, 8)
                pltpu.sync_copy(acc2_hbm.at[pl.ds(base, NR)], grp)
                j0 = jnp.maximum(n0 - base, 0)
                j1 = jnp.minimum(n1 - base, NR)

                def row_body(j, _):
                    for q in range(REC2 // 16):
                        sl = pl.ds(q * 16, 16)
                        xr = jnp.maximum(grp[j, sl] + bv[sl], 0.0)
                        maxa[sl] = jnp.maximum(maxa[sl], xr)
                        plsc.addupdate(suma.at[sl], xr)
                    return 0

                return lax.fori_loop(j0, j1, row_body, 0)

            nch = (n1 + NR - 1) // NR - ka0
            lax.fori_loop(0, nch, chunk_body, 0)
            cntf = jnp.maximum(n1 - n0, 1).astype(jnp.float32)
            for q in range(REC2 // 16):
                sl = pl.ds(q * 16, 16)
                pstage[s, sl] = maxa[sl]
                pstage[s, pl.ds(REC2 + q * 16, 16)] = suma[sl] / cntf
            return 0

        lax.fori_loop(0, SEGT, seg_body, 0)
        pltpu.sync_copy(pstage, out_hbm.at[pl.ds(s0, SEGT)])

    return pool


def kernel(x1, edge_index1, batch1, cell, x2, edge_index2, batch2,
           gat_W, gat_as, gat_ad, gat_b, gcn_W, gcn_b,
           g1_W, g1_b, g2_W, g2_b, g3_W, g3_b,
           r1_W, r1_b, r2_W, r2_b, r3_W, r3_b,
           f1_W, f1_b, f2_W, f2_b, o_W, o_b):
    f32, i32 = jnp.float32, jnp.int32
    N, C = x1.shape
    H = gat_as.shape[0]
    F = H * C
    CP = ((C + 15) // 16) * 16
    REC1 = 16 + H * CP
    REC2 = ((F + 19) // 16) * 16
    Bb = cell.shape[0]
    BT = 2 * Bb
    NT = 2 * N
    NPT = (((NT + TILES - 1) // TILES) + 7) // 8 * 8
    N1 = NPT * TILES
    assert N1 % 256 == 0 and NT % RG == 0
    E = edge_index1.shape[1]
    ET = 2 * (E + N)

    loops = jnp.arange(N, dtype=i32)
    src = jnp.concatenate([edge_index1[0], loops, edge_index2[0] + N, loops + N])
    dst = jnp.concatenate([edge_index1[1], loops, edge_index2[1] + N, loops + N])
    dst_s, src_s = lax.sort((dst, src), num_keys=1)
    row_ptr = jnp.searchsorted(dst_s, jnp.arange(NT + 1, dtype=i32),
                               side='left').astype(i32)
    rp_pad = jnp.concatenate([row_ptr, jnp.full((N1 + 8 - (NT + 1),), ET, i32)])
    src_pad = jnp.concatenate([src_s, jnp.zeros((EG + K,), i32)])
    dst_pad = jnp.concatenate([dst_s, jnp.full((EG + K,), N1, i32)])
    deg = jnp.diff(row_ptr).astype(f32)
    deg_col = jnp.concatenate([deg, jnp.ones((N1 - NT,), f32)]).reshape(N1, 1)
    bt = jnp.concatenate([batch1, batch2 + Bb])
    bp = jnp.searchsorted(bt, jnp.arange(BT + 1, dtype=i32),
                          side='left').astype(i32)
    bp_pad = jnp.concatenate([bp, jnp.full((7,), NT, i32)])

    Ws = jnp.einsum('chd,hd->ch', gat_W.reshape(C, H, C), gat_as)
    Wd = jnp.einsum('chd,hd->ch', gat_W.reshape(C, H, C), gat_ad)
    W_hp = jnp.pad(gat_W.reshape(C, H, C), ((0, 0), (0, 0), (0, CP - C))
                   ).reshape(C, H * CP)
    W_all = jnp.concatenate([Ws, jnp.zeros((C, 16 - H), f32), W_hp], axis=1)
    Wd_pad = jnp.concatenate([Wd, jnp.zeros((C, 16 - H), f32)], axis=1)
    b1p = jnp.concatenate([jnp.zeros((16,), f32),
                           jnp.pad(gat_b.reshape(H, C), ((0, 0), (0, CP - C))
                                   ).reshape(H * CP)]).reshape(1, REC1)
    Wg = jnp.pad(gcn_W.reshape(H, C, F), ((0, 0), (0, CP - C), (0, 0))
                 ).reshape(H * CP, F)
    Wg_ext = jnp.concatenate([jnp.zeros((16, F), f32), Wg], axis=0)
    Wg_ext = jnp.pad(Wg_ext, ((0, 0), (0, REC2 - F)))
    b2p = jnp.pad(gcn_b, (0, REC2 - F))
    g1p = jnp.concatenate([g1_W[:F], jnp.zeros((REC2 - F, 512), f32),
                           g1_W[F:], jnp.zeros((REC2 - F, 512), f32)], axis=0)

    Xp = jnp.pad(jnp.concatenate([x1, x2], axis=0), ((0, N1 - NT), (0, 0)))

    rec1, ad = pl.pallas_call(
        _s1_body,
        grid=(N1 // 256,),
        in_specs=[pl.BlockSpec((256, C), lambda i: (i, 0)),
                  pl.BlockSpec((C, REC1), lambda i: (0, 0)),
                  pl.BlockSpec((C, 16), lambda i: (0, 0))],
        out_specs=[pl.BlockSpec((256, REC1), lambda i: (i, 0)),
                   pl.BlockSpec((256, 16), lambda i: (i, 0))],
        out_shape=[jax.ShapeDtypeStruct((N1, REC1), f32),
                   jax.ShapeDtypeStruct((N1, 16), f32)],
    )(Xp, W_all, Wd_pad)

    gat = _make_gat_agg(N1, NPT, REC1, H, CP)(rec1, ad, rp_pad, src_pad, dst_pad)

    rec2, dinv = pl.pallas_call(
        _s3_body,
        grid=(N1 // 256,),
        in_specs=[pl.BlockSpec((256, REC1), lambda i: (i, 0)),
                  pl.BlockSpec((256, 1), lambda i: (i, 0)),
                  pl.BlockSpec((REC1, REC2), lambda i: (0, 0)),
                  pl.BlockSpec((1, REC1), lambda i: (0, 0))],
        out_specs=[pl.BlockSpec((256, REC2), lambda i: (i, 0)),
                   pl.BlockSpec((256, 1), lambda i: (i, 0))],
        out_shape=[jax.ShapeDtypeStruct((N1, REC2), f32),
                   jax.ShapeDtypeStruct((N1, 1), f32)],
    )(gat, deg_col, Wg_ext, b1p)

    acc2 = _make_gcn_agg(N1, NPT, REC2)(
        rec2, dinv.reshape(N1), rp_pad, src_pad, dst_pad)

    pooled = _make_pool(N1, BT, REC2)(acc2, b2p, bp_pad)

    def b2(v):
        return v.reshape(1, -1)

    out = pl.pallas_call(
        _s6_body,
        out_shape=jax.ShapeDtypeStruct((Bb, 2), f32),
    )(pooled, cell, g1p, b2(g1_b), g2_W, b2(g2_b), g3_W, b2(g3_b),
      r1_W, b2(r1_b), r2_W, b2(r2_b), r3_W, b2(r3_b),
      f1_W, b2(f1_b), f2_W, b2(f2_b), o_W, b2(o_b))
    return out


# speedup vs baseline: 3.0409x; 3.0409x over previous
import functools

import jax
import jax.numpy as jnp
from jax import lax
from jax.experimental import pallas as pl
from jax.experimental.pallas import tpu as pltpu
from jax.experimental.pallas import tpu_sc as plsc

NC = 2
NS = 16
TILES = NC * NS

K = 32
EG = 2048
RG = 8


def _s1_body(x_ref, wall_ref, wd_ref, rec_ref, ad_ref):
    x = x_ref[...]
    rec_ref[...] = jnp.dot(x, wall_ref[...], preferred_element_type=jnp.float32)
    ad_ref[...] = jnp.dot(x, wd_ref[...], preferred_element_type=jnp.float32)


def _s3_body(acc_ref, wg_ref, b_ref, rec2_ref, dinv_ref):
    a = acc_ref[...]
    h = jnp.maximum(a + b_ref[...], 0.0)
    dinv = lax.rsqrt(a[:, 10:11])
    mm = jnp.dot(h, wg_ref[...], preferred_element_type=jnp.float32)
    rec2_ref[...] = mm * dinv
    dinv_ref[...] = dinv


def _s6_body(pooled_ref, cell_ref,
             g1_ref, g1b_ref, g2_ref, g2b_ref, g3_ref, g3b_ref,
             r1_ref, r1b_ref, r2_ref, r2b_ref, r3_ref, r3b_ref,
             f1_ref, f1b_ref, f2_ref, f2b_ref, ow_ref, ob_ref,
             out_ref):
    f32 = jnp.float32
    y = jnp.maximum(jnp.dot(pooled_ref[...], g1_ref[...], preferred_element_type=f32)
                    + g1b_ref[...], 0.0)
    y = jnp.dot(y, g2_ref[...], preferred_element_type=f32) + g2b_ref[...]
    y = jnp.dot(y, g3_ref[...], preferred_element_type=f32) + g3b_ref[...]
    nb = out_ref.shape[0]
    h1 = y[:nb]
    h2 = y[nb:2 * nb]
    cv = cell_ref[...]
    nrm = jnp.sqrt(jnp.sum(cv * cv, axis=1, keepdims=True))
    cv = cv / jnp.maximum(nrm, 1e-12)
    cv = jnp.maximum(jnp.dot(cv, r1_ref[...], preferred_element_type=f32)
                     + r1b_ref[...], 0.0)
    cv = jnp.maximum(jnp.dot(cv, r2_ref[...], preferred_element_type=f32)
                     + r2b_ref[...], 0.0)
    cv = jnp.dot(cv, r3_ref[...], preferred_element_type=f32) + r3b_ref[...]
    xc = jnp.concatenate([h1, h2, cv], axis=1)
    xc = jnp.maximum(jnp.dot(xc, f1_ref[...], preferred_element_type=f32)
                     + f1b_ref[...], 0.0)
    xc = jnp.maximum(jnp.dot(xc, f2_ref[...], preferred_element_type=f32)
                     + f2b_ref[...], 0.0)
    out_ref[...] = jnp.dot(xc, ow_ref[...], preferred_element_type=f32) + ob_ref[...]


def _make_gat_agg(N1, NPT, REC1, H, CP):
    mesh = plsc.VectorSubcoreMesh(core_axis_name="c", subcore_axis_name="s")

    @functools.partial(
        pl.kernel, mesh=mesh,
        compiler_params=pltpu.CompilerParams(use_tc_tiling_on_sc=False),
        out_type=jax.ShapeDtypeStruct((N1, REC1), jnp.float32),
        scratch_types=[
            pltpu.VMEM((EG + 16,), jnp.int32),
            pltpu.VMEM((EG + 16,), jnp.int32),
            pltpu.VMEM((K, REC1), jnp.float32),
            pltpu.VMEM((K, REC1), jnp.float32),
            pltpu.VMEM((NPT, 16), jnp.float32),
            pltpu.VMEM((48,), jnp.int32),
            pltpu.VMEM((REC1,), jnp.float32),
            pltpu.VMEM((16,), jnp.float32),
            pltpu.VMEM((RG, REC1), jnp.float32),
            pltpu.SemaphoreType.DMA,
            pltpu.SemaphoreType.DMA,
        ])
    def gat_agg(rec_hbm, ad_hbm, tp_hbm, src_hbm, dst_hbm, out_hbm,
                srcv, dstv, gbuf0, gbuf1, adv, tpv, acc, wsum, stage,
                sem0, sem1):
        t = lax.axis_index("s") * NC + lax.axis_index("c")
        ns = pl.multiple_of(t * NPT, 8)
        pltpu.sync_copy(tp_hbm.at[pl.ds(0, 48)], tpv)
        pltpu.sync_copy(ad_hbm.at[pl.ds(ns, NPT)], adv)
        e0 = tpv[pl.ds(t, 16)][0]
        e1 = tpv[pl.ds(t + 1, 16)][0]
        a0 = pl.multiple_of((e0 // K) * K, 8)
        ngr = (e1 - a0 + (EG - 1)) // EG

        zero16 = jnp.zeros((16,), jnp.float32)
        for q in range(REC1 // 16):
            acc[pl.ds(q * 16, 16)] = zero16
        wsum[...] = zero16

        def flush(cur):
            r = cur - ns
            slot = r - (r // RG) * RG
            ws = wsum[...]
            for hh in range(H):
                s = ws[hh] + 1e-16
                for j5 in range(CP // 16):
                    off = 16 + hh * CP + j5 * 16
                    stage[slot, pl.ds(off, 16)] = acc[pl.ds(off, 16)] / s
            stage[slot, pl.ds(0, 16)] = ws

            @pl.when(slot == RG - 1)
            def _():
                pltpu.sync_copy(
                    stage, out_hbm.at[pl.ds(pl.multiple_of(cur - (RG - 1), 8), RG)])

        def process(ci, gbuf, cur):
            def edge_body(j, carry):
                cur, ci = carry
                d = dstv[pl.ds(ci * K + j, 16)][0]
                trans = d != cur

                @pl.when(jnp.logical_and(
                    trans, jnp.logical_and(cur >= ns, cur < ns + NPT)))
                def _():
                    flush(cur)

                @pl.when(trans)
                def _():
                    for q in range(REC1 // 16):
                        acc[pl.ds(q * 16, 16)] = zero16
                    wsum[...] = zero16

                cur = jnp.where(trans, d, cur)
                ia = jnp.minimum(d - ns, NPT - 1)
                av = gbuf[j, pl.ds(0, 16)] + adv[ia]
                al = jnp.maximum(av, 0.2 * av)
                w = jnp.exp(al)
                wsum[...] = wsum[...] + w
                for hh in range(H):
                    wh = w[hh]
                    for j5 in range(CP // 16):
                        off = 16 + hh * CP + j5 * 16
                        plsc.addupdate(acc.at[pl.ds(off, 16)],
                                       wh * gbuf[j, pl.ds(off, 16)])
                return cur, ci

            cur, _ = lax.fori_loop(0, K, edge_body, (cur, ci))
            return cur

        def group_body(g, cur):
            gstart = pl.multiple_of(a0 + g * EG, 8)
            pltpu.sync_copy(src_hbm.at[pl.ds(gstart, EG)], srcv.at[pl.ds(0, EG)])
            pltpu.sync_copy(dst_hbm.at[pl.ds(gstart, EG)], dstv.at[pl.ds(0, EG)])
            ncg = jnp.minimum((e1 - gstart + (K - 1)) // K, EG // K)
            pltpu.async_copy(rec_hbm.at[srcv.at[pl.ds(0, K)]], gbuf0, sem0)

            def pair(ci2, cur):
                base = ci2 * 2

                @pl.when(base + 1 < ncg)
                def _():
                    pltpu.async_copy(
                        rec_hbm.at[srcv.at[pl.ds((base + 1) * K, K)]], gbuf1, sem1)

                pltpu.make_async_copy(rec_hbm.at[pl.ds(0, K)], gbuf0, sem0).wait()
                cur = process(base, gbuf0, cur)

                @pl.when(base + 2 < ncg)
                def _():
                    pltpu.async_copy(
                        rec_hbm.at[srcv.at[pl.ds((base + 2) * K, K)]], gbuf0, sem0)

                def do1(c):
                    pltpu.make_async_copy(
                        rec_hbm.at[pl.ds(0, K)], gbuf1, sem1).wait()
                    return process(base + 1, gbuf1, c)

                return lax.cond(base + 1 < ncg, do1, lambda c: c, cur)

            return lax.fori_loop(0, (ncg + 1) // 2, pair, cur)

        cur = lax.fori_loop(0, ngr, group_body, jnp.int32(-1))

        @pl.when(jnp.logical_and(cur >= ns, cur < ns + NPT))
        def _():
            flush(cur)

    return gat_agg


def _make_gcn_agg(N1, NPT, REC2):
    mesh = plsc.VectorSubcoreMesh(core_axis_name="c", subcore_axis_name="s")

    @functools.partial(
        pl.kernel, mesh=mesh,
        compiler_params=pltpu.CompilerParams(use_tc_tiling_on_sc=False),
        out_type=jax.ShapeDtypeStruct((N1, REC2), jnp.float32),
        scratch_types=[
            pltpu.VMEM((EG + 16,), jnp.int32),
            pltpu.VMEM((EG + 16,), jnp.int32),
            pltpu.VMEM((K, REC2), jnp.float32),
            pltpu.VMEM((K, REC2), jnp.float32),
            pltpu.VMEM((NPT + 16,), jnp.float32),
            pltpu.VMEM((48,), jnp.int32),
            pltpu.VMEM((REC2,), jnp.float32),
            pltpu.VMEM((RG, REC2), jnp.float32),
            pltpu.SemaphoreType.DMA,
            pltpu.SemaphoreType.DMA,
        ])
    def gcn_agg(rec_hbm, dinv_hbm, tp_hbm, src_hbm, dst_hbm, out_hbm,
                srcv, dstv, gbuf0, gbuf1, dvv, tpv, acc, stage, sem0, sem1):
        t = lax.axis_index("s") * NC + lax.axis_index("c")
        ns = pl.multiple_of(t * NPT, 8)
        pltpu.sync_copy(tp_hbm.at[pl.ds(0, 48)], tpv)
        pltpu.sync_copy(dinv_hbm.at[pl.ds(ns, NPT)], dvv.at[pl.ds(0, NPT)])
        e0 = tpv[pl.ds(t, 16)][0]
        e1 = tpv[pl.ds(t + 1, 16)][0]
        a0 = pl.multiple_of((e0 // K) * K, 8)
        ngr = (e1 - a0 + (EG - 1)) // EG

        zero16 = jnp.zeros((16,), jnp.float32)
        for q in range(REC2 // 16):
            acc[pl.ds(q * 16, 16)] = zero16

        def flush(cur):
            r = cur - ns
            slot = r - (r // RG) * RG
            for q in range(REC2 // 16):
                stage[slot, pl.ds(q * 16, 16)] = acc[pl.ds(q * 16, 16)]

            @pl.when(slot == RG - 1)
            def _():
                pltpu.sync_copy(
                    stage, out_hbm.at[pl.ds(pl.multiple_of(cur - (RG - 1), 8), RG)])

        def process(ci, gbuf, cur):
            def edge_body(j, carry):
                cur, ci = carry
                d = dstv[pl.ds(ci * K + j, 16)][0]
                trans = d != cur

                @pl.when(jnp.logical_and(
                    trans, jnp.logical_and(cur >= ns, cur < ns + NPT)))
                def _():
                    flush(cur)

                @pl.when(trans)
                def _():
                    for q in range(REC2 // 16):
                        acc[pl.ds(q * 16, 16)] = zero16

                cur = jnp.where(trans, d, cur)
                iv = jnp.minimum(d - ns, NPT - 1)
                dv = dvv[pl.ds(iv, 16)][0]
                for q in range(REC2 // 16):
                    plsc.addupdate(acc.at[pl.ds(q * 16, 16)],
                                   dv * gbuf[j, pl.ds(q * 16, 16)])
                return cur, ci

            cur, _ = lax.fori_loop(0, K, edge_body, (cur, ci))
            return cur

        def group_body(g, cur):
            gstart = pl.multiple_of(a0 + g * EG, 8)
            pltpu.sync_copy(src_hbm.at[pl.ds(gstart, EG)], srcv.at[pl.ds(0, EG)])
            pltpu.sync_copy(dst_hbm.at[pl.ds(gstart, EG)], dstv.at[pl.ds(0, EG)])
            ncg = jnp.minimum((e1 - gstart + (K - 1)) // K, EG // K)
            pltpu.async_copy(rec_hbm.at[srcv.at[pl.ds(0, K)]], gbuf0, sem0)

            def pair(ci2, cur):
                base = ci2 * 2

                @pl.when(base + 1 < ncg)
                def _():
                    pltpu.async_copy(
                        rec_hbm.at[srcv.at[pl.ds((base + 1) * K, K)]], gbuf1, sem1)

                pltpu.make_async_copy(rec_hbm.at[pl.ds(0, K)], gbuf0, sem0).wait()
                cur = process(base, gbuf0, cur)

                @pl.when(base + 2 < ncg)
                def _():
                    pltpu.async_copy(
                        rec_hbm.at[srcv.at[pl.ds((base + 2) * K, K)]], gbuf0, sem0)

                def do1(c):
                    pltpu.make_async_copy(
                        rec_hbm.at[pl.ds(0, K)], gbuf1, sem1).wait()
                    return process(base + 1, gbuf1, c)

                return lax.cond(base + 1 < ncg, do1, lambda c: c, cur)

            return lax.fori_loop(0, (ncg + 1) // 2, pair, cur)

        cur = lax.fori_loop(0, ngr, group_body, jnp.int32(-1))

        @pl.when(jnp.logical_and(cur >= ns, cur < ns + NPT))
        def _():
            flush(cur)

    return gcn_agg


def _make_pool(N1, BT, REC2):
    mesh = plsc.VectorSubcoreMesh(core_axis_name="c", subcore_axis_name="s")
    SEGT = BT // TILES
    NR = 8

    @functools.partial(
        pl.kernel, mesh=mesh,
        compiler_params=pltpu.CompilerParams(use_tc_tiling_on_sc=False),
        out_type=jax.ShapeDtypeStruct((BT, 2 * REC2), jnp.float32),
        scratch_types=[
            pltpu.VMEM((SEGT + 24,), jnp.int32),
            pltpu.VMEM((NR, REC2), jnp.float32),
            pltpu.VMEM((REC2,), jnp.float32),
            pltpu.VMEM((REC2,), jnp.float32),
            pltpu.VMEM((REC2,), jnp.float32),
            pltpu.VMEM((SEGT, 2 * REC2), jnp.float32),
        ])
    def pool(acc2_hbm, b_hbm, bp_hbm, out_hbm,
             bpv, grp, bv, maxa, suma, pstage):
        t = lax.axis_index("s") * NC + lax.axis_index("c")
        s0 = pl.multiple_of(t * SEGT, 8)
        pltpu.sync_copy(bp_hbm.at[pl.ds(s0, SEGT + 8)], bpv.at[pl.ds(0, SEGT + 8)])
        pltpu.sync_copy(b_hbm, bv)
        zero16 = jnp.zeros((16,), jnp.float32)

        def seg_body(s, _):
            n0 = bpv[pl.ds(s, 16)][0]
            n1 = bpv[pl.ds(s + 1, 16)][0]
            for q in range(REC2 // 16):
                maxa[pl.ds(q * 16, 16)] = zero16
                suma[pl.ds(q * 16, 16)] = zero16

            ka0 = n0 // NR

            def chunk_body(kk, _):
                base = pl.multiple_of((ka0 + kk) * NR, 8)
                pltpu.sync_copy(acc2_hbm.at[pl.ds(base, NR)], grp)
                j0 = jnp.maximum(n0 - base, 0)
                j1 = jnp.minimum(n1 - base, NR)

                def row_body(j, _):
                    for q in range(REC2 // 16):
                        sl = pl.ds(q * 16, 16)
                        xr = jnp.maximum(grp[j, sl] + bv[sl], 0.0)
                        maxa[sl] = jnp.maximum(maxa[sl], xr)
                        plsc.addupdate(suma.at[sl], xr)
                    return 0

                return lax.fori_loop(j0, j1, row_body, 0)

            nch = (n1 + NR - 1) // NR - ka0
            lax.fori_loop(0, nch, chunk_body, 0)
            cntf = jnp.maximum(n1 - n0, 1).astype(jnp.float32)
            for q in range(REC2 // 16):
                sl = pl.ds(q * 16, 16)
                pstage[s, sl] = maxa[sl]
                pstage[s, pl.ds(REC2 + q * 16, 16)] = suma[sl] / cntf
            return 0

        lax.fori_loop(0, SEGT, seg_body, 0)
        pltpu.sync_copy(pstage, out_hbm.at[pl.ds(s0, SEGT)])

    return pool


def kernel(x1, edge_index1, batch1, cell, x2, edge_index2, batch2,
           gat_W, gat_as, gat_ad, gat_b, gcn_W, gcn_b,
           g1_W, g1_b, g2_W, g2_b, g3_W, g3_b,
           r1_W, r1_b, r2_W, r2_b, r3_W, r3_b,
           f1_W, f1_b, f2_W, f2_b, o_W, o_b):
    f32, i32 = jnp.float32, jnp.int32
    N, C = x1.shape
    H = gat_as.shape[0]
    F = H * C
    CP = ((C + 15) // 16) * 16
    REC1 = 16 + H * CP
    REC2 = ((F + 19) // 16) * 16
    Bb = cell.shape[0]
    BT = 2 * Bb
    NT = 2 * N
    NPT = (((NT + TILES - 1) // TILES) + 7) // 8 * 8
    N1 = NPT * TILES
    assert N1 % 256 == 0 and NT % RG == 0
    E = edge_index1.shape[1]
    ET = 2 * (E + N)

    loops = jnp.arange(N, dtype=i32)
    src = jnp.concatenate([edge_index1[0], loops, edge_index2[0] + N, loops + N])
    dst = jnp.concatenate([edge_index1[1], loops, edge_index2[1] + N, loops + N])
    dst_s, src_s = lax.sort((dst, src), num_keys=1)
    tile_ptr = jnp.searchsorted(
        dst_s, jnp.arange(0, N1 + 1, NPT, dtype=i32), side='left').astype(i32)
    tp_pad = jnp.concatenate([tile_ptr, jnp.full((48 - (TILES + 1),), ET, i32)])
    src_pad = jnp.concatenate([src_s, jnp.zeros((EG + K,), i32)])
    dst_pad = jnp.concatenate([dst_s, jnp.full((EG + K,), N1, i32)])
    bt = jnp.concatenate([batch1, batch2 + Bb])
    bp = jnp.searchsorted(bt, jnp.arange(BT + 1, dtype=i32),
                          side='left').astype(i32)
    bp_pad = jnp.concatenate([bp, jnp.full((7,), NT, i32)])

    Ws = jnp.einsum('chd,hd->ch', gat_W.reshape(C, H, C), gat_as)
    Wd = jnp.einsum('chd,hd->ch', gat_W.reshape(C, H, C), gat_ad)
    W_hp = jnp.pad(gat_W.reshape(C, H, C), ((0, 0), (0, 0), (0, CP - C))
                   ).reshape(C, H * CP)
    W_all = jnp.concatenate([Ws, jnp.zeros((C, 16 - H), f32), W_hp], axis=1)
    Wd_pad = jnp.concatenate([Wd, jnp.zeros((C, 16 - H), f32)], axis=1)
    b1p = jnp.concatenate([jnp.zeros((16,), f32),
                           jnp.pad(gat_b.reshape(H, C), ((0, 0), (0, CP - C))
                                   ).reshape(H * CP)]).reshape(1, REC1)
    Wg = jnp.pad(gcn_W.reshape(H, C, F), ((0, 0), (0, CP - C), (0, 0))
                 ).reshape(H * CP, F)
    Wg_ext = jnp.concatenate([jnp.zeros((16, F), f32), Wg], axis=0)
    Wg_ext = jnp.pad(Wg_ext, ((0, 0), (0, REC2 - F)))
    b2p = jnp.pad(gcn_b, (0, REC2 - F))
    g1p = jnp.concatenate([g1_W[:F], jnp.zeros((REC2 - F, 512), f32),
                           g1_W[F:], jnp.zeros((REC2 - F, 512), f32)], axis=0)

    Xp = jnp.pad(jnp.concatenate([x1, x2], axis=0), ((0, N1 - NT), (0, 0)))

    rec1, ad = pl.pallas_call(
        _s1_body,
        grid=(N1 // 256,),
        in_specs=[pl.BlockSpec((256, C), lambda i: (i, 0)),
                  pl.BlockSpec((C, REC1), lambda i: (0, 0)),
                  pl.BlockSpec((C, 16), lambda i: (0, 0))],
        out_specs=[pl.BlockSpec((256, REC1), lambda i: (i, 0)),
                   pl.BlockSpec((256, 16), lambda i: (i, 0))],
        out_shape=[jax.ShapeDtypeStruct((N1, REC1), f32),
                   jax.ShapeDtypeStruct((N1, 16), f32)],
    )(Xp, W_all, Wd_pad)

    gat = _make_gat_agg(N1, NPT, REC1, H, CP)(rec1, ad, tp_pad, src_pad, dst_pad)

    rec2, dinv = pl.pallas_call(
        _s3_body,
        grid=(N1 // 256,),
        in_specs=[pl.BlockSpec((256, REC1), lambda i: (i, 0)),
                  pl.BlockSpec((REC1, REC2), lambda i: (0, 0)),
                  pl.BlockSpec((1, REC1), lambda i: (0, 0))],
        out_specs=[pl.BlockSpec((256, REC2), lambda i: (i, 0)),
                   pl.BlockSpec((256, 1), lambda i: (i, 0))],
        out_shape=[jax.ShapeDtypeStruct((N1, REC2), f32),
                   jax.ShapeDtypeStruct((N1, 1), f32)],
    )(gat, Wg_ext, b1p)

    acc2 = _make_gcn_agg(N1, NPT, REC2)(
        rec2, dinv.reshape(N1), tp_pad, src_pad, dst_pad)

    pooled = _make_pool(N1, BT, REC2)(acc2, b2p, bp_pad)

    def b2(v):
        return v.reshape(1, -1)

    out = pl.pallas_call(
        _s6_body,
        out_shape=jax.ShapeDtypeStruct((Bb, 2), f32),
    )(pooled, cell, g1p, b2(g1_b), g2_W, b2(g2_b), g3_W, b2(g3_b),
      r1_W, b2(r1_b), r2_W, b2(r2_b), r3_W, b2(r3_b),
      f1_W, b2(f1_b), f2_W, b2(f2_b), o_W, b2(o_b))
    return out
